# Initial kernel scaffold; baseline (speedup 1.0000x reference)
import jax
import jax.numpy as jnp
from jax import lax
from jax.experimental import pallas as pl
from jax.experimental.pallas import tpu as pltpu
from jax.experimental.pallas import tpu_sc as plsc

N_NODES = 10000
N_EDGES = 160000
D = 256
H = 128
NC = 2
NS = 16
ET = N_EDGES // NS
CHUNK = 80
NCHUNK = ET // CHUNK
ROWS_PER_TILE = N_NODES // NS
FLUSH = 125
NFLUSH = ROWS_PER_TILE // FLUSH

NB = 1000
GRID = N_NODES // NB


def _matmul_body(ft_ref, ws_ref, wo_ref, bs_ref, bo_ref, fst_ref, fot_ref):
    x = ft_ref[...]
    ys = jnp.dot(x, ws_ref[...], preferred_element_type=jnp.float32)
    ys = ys + bs_ref[...]
    yo = jnp.dot(x, wo_ref[...], preferred_element_type=jnp.float32)
    yo = yo + bo_ref[...]
    fst_ref[0] = ys[:, :H]
    fst_ref[1] = ys[:, H:]
    fot_ref[0] = yo[:, :H]
    fot_ref[1] = yo[:, H:]


def _tc_matmuls(ft, wst, wot, bs, bo):
    return pl.pallas_call(
        _matmul_body,
        grid=(GRID,),
        in_specs=[
            pl.BlockSpec((NB, D), lambda i: (i, 0)),
            pl.BlockSpec((D, D), lambda i: (0, 0)),
            pl.BlockSpec((D, D), lambda i: (0, 0)),
            pl.BlockSpec((1, D), lambda i: (0, 0)),
            pl.BlockSpec((1, D), lambda i: (0, 0)),
        ],
        out_specs=[
            pl.BlockSpec((NC, NB, H), lambda i: (0, i, 0)),
            pl.BlockSpec((NC, NB, H), lambda i: (0, i, 0)),
        ],
        out_shape=[
            jax.ShapeDtypeStruct((NC, N_NODES, H), jnp.float32),
            jax.ShapeDtypeStruct((NC, N_NODES, H), jnp.float32),
        ],
    )(ft, wst, wot, bs, bo)


def _sc_body(fot, fst, gidx, ridx, out, idxg_v, idxr_v, rows_v, buf_v,
             acc_sh, sem):
    c = lax.axis_index("c")
    s = lax.axis_index("s")
    for k in range(NFLUSH):
        r0 = s * ROWS_PER_TILE + k * FLUSH
        pltpu.sync_copy(fst.at[c, pl.ds(r0, FLUSH)], buf_v)
        pltpu.sync_copy(buf_v, acc_sh.at[pl.ds(r0, FLUSH)])
    pltpu.sync_copy(gidx.at[c, s], idxg_v)
    pltpu.sync_copy(ridx.at[s], idxr_v)
    plsc.subcore_barrier()

    def edge_chunk(j, carry):
        pltpu.async_copy(fot.at[idxg_v.at[j]], rows_v, sem).wait()
        pltpu.sync_copy(rows_v, acc_sh.at[idxr_v.at[j]], add=True)
        return carry

    lax.fori_loop(0, NCHUNK, edge_chunk, 0)
    plsc.subcore_barrier()
    for k in range(NFLUSH):
        r0 = s * ROWS_PER_TILE + k * FLUSH
        pltpu.sync_copy(acc_sh.at[pl.ds(r0, FLUSH)], buf_v)
        pltpu.sync_copy(buf_v, out.at[c, pl.ds(r0, FLUSH)])


def _sc_aggregate(fot_flat, fst, gidx, ridx):
    mesh = plsc.VectorSubcoreMesh(core_axis_name="c", subcore_axis_name="s")
    return pl.kernel(
        _sc_body,
        out_type=jax.ShapeDtypeStruct((NC, N_NODES, H), jnp.float32),
        mesh=mesh,
        scratch_types=[
            pltpu.VMEM((NCHUNK, CHUNK), jnp.int32),
            pltpu.VMEM((NCHUNK, CHUNK), jnp.int32),
            pltpu.VMEM((CHUNK, H), jnp.float32),
            pltpu.VMEM((FLUSH, H), jnp.float32),
            pltpu.VMEM_SHARED((N_NODES, H), jnp.float32),
            pltpu.SemaphoreType.DMA,
        ],
    )(fot_flat, fst, gidx, ridx)


def kernel(features, w_self, b_self, w_other, b_other, reduce_index,
           gather_index):
    ft = features[0].T
    wst = w_self.T
    wot = w_other.T
    bs = b_self[None, :]
    bo = b_other[None, :]
    gi = gather_index.astype(jnp.int32)
    ri = reduce_index.astype(jnp.int32)
    gidx = jnp.stack([gi, gi + N_NODES]).reshape(NC, NS, NCHUNK, CHUNK)
    ridx = ri.reshape(NS, NCHUNK, CHUNK)

    fst, fot = _tc_matmuls(ft, wst, wot, bs, bo)
    fot_flat = fot.reshape(NC * N_NODES, H)
    acc = _sc_aggregate(fot_flat, fst, gidx, ridx)
    return jnp.transpose(acc, (0, 2, 1)).reshape(1, D, N_NODES)


# speedup vs baseline: 3.3429x; 3.3429x over previous
import jax
import jax.numpy as jnp
from jax import lax
from jax.experimental import pallas as pl
from jax.experimental.pallas import tpu as pltpu
from jax.experimental.pallas import tpu_sc as plsc

N_NODES = 10000
N_EDGES = 160000
D = 256
H = 128
NC = 2
NS = 16
ET = N_EDGES // NS
CHUNK = 80
NCHUNK = ET // CHUNK
NP = 10240
ROWS_PER_TILE = NP // NS
FLUSH = 128
NFLUSH = ROWS_PER_TILE // FLUSH

NB = 1000
GRID = N_NODES // NB


def _matmul_body(ft_ref, ws_ref, wo_ref, bs_ref, bo_ref, fst_ref, fot_ref):
    x = ft_ref[...]
    ys = jnp.dot(x, ws_ref[...], preferred_element_type=jnp.float32)
    ys = ys + bs_ref[...]
    yo = jnp.dot(x, wo_ref[...], preferred_element_type=jnp.float32)
    yo = yo + bo_ref[...]
    fst_ref[0] = ys[:, :H]
    fst_ref[1] = ys[:, H:]
    fot_ref[0] = yo[:, :H]
    fot_ref[1] = yo[:, H:]


def _tc_matmuls(ft, wst, wot, bs, bo):
    return pl.pallas_call(
        _matmul_body,
        grid=(GRID,),
        in_specs=[
            pl.BlockSpec((NB, D), lambda i: (i, 0)),
            pl.BlockSpec((D, D), lambda i: (0, 0)),
            pl.BlockSpec((D, D), lambda i: (0, 0)),
            pl.BlockSpec((1, D), lambda i: (0, 0)),
            pl.BlockSpec((1, D), lambda i: (0, 0)),
        ],
        out_specs=[
            pl.BlockSpec((NC, NB, H), lambda i: (0, i, 0)),
            pl.BlockSpec((NC, NB, H), lambda i: (0, i, 0)),
        ],
        out_shape=[
            jax.ShapeDtypeStruct((NC, NP, H), jnp.float32),
            jax.ShapeDtypeStruct((NC, N_NODES, H), jnp.float32),
        ],
    )(ft, wst, wot, bs, bo)


def _sc_body(fot, fst, gidx, ridx, out, g80_v, r80_v, rows_v, buf_v,
             acc_sh, sem):
    c = lax.axis_index("c")
    s = lax.axis_index("s")
    for k in range(NFLUSH):
        r0 = s * ROWS_PER_TILE + k * FLUSH
        pltpu.sync_copy(fst.at[c, pl.ds(r0, FLUSH)], buf_v)
        pltpu.sync_copy(buf_v, acc_sh.at[pl.ds(r0, FLUSH)])
    plsc.subcore_barrier()
    g_base = c * N_EDGES + s * ET
    r_base = s * ET

    def edge_chunk(j, carry):
        pltpu.sync_copy(gidx.at[pl.ds(g_base + j * CHUNK, CHUNK)], g80_v)
        pltpu.sync_copy(ridx.at[pl.ds(r_base + j * CHUNK, CHUNK)], r80_v)
        pltpu.async_copy(fot.at[g80_v], rows_v, sem).wait()
        pltpu.sync_copy(rows_v, acc_sh.at[r80_v], add=True)
        return carry

    lax.fori_loop(0, NCHUNK, edge_chunk, 0)
    plsc.subcore_barrier()
    for k in range(NFLUSH):
        r0 = s * ROWS_PER_TILE + k * FLUSH
        pltpu.sync_copy(acc_sh.at[pl.ds(r0, FLUSH)], buf_v)
        pltpu.sync_copy(buf_v, out.at[c, pl.ds(r0, FLUSH)])


def _sc_aggregate(fot_flat, fst, gidx, ridx):
    mesh = plsc.VectorSubcoreMesh(core_axis_name="c", subcore_axis_name="s")
    return pl.kernel(
        _sc_body,
        out_type=jax.ShapeDtypeStruct((NC, NP, H), jnp.float32),
        mesh=mesh,
        scratch_types=[
            pltpu.VMEM((CHUNK,), jnp.int32),
            pltpu.VMEM((CHUNK,), jnp.int32),
            pltpu.VMEM((CHUNK, H), jnp.float32),
            pltpu.VMEM((FLUSH, H), jnp.float32),
            pltpu.VMEM_SHARED((NP, H), jnp.float32),
            pltpu.SemaphoreType.DMA,
        ],
    )(fot_flat, fst, gidx, ridx)


def kernel(features, w_self, b_self, w_other, b_other, reduce_index,
           gather_index):
    ft = features[0].T
    wst = w_self.T
    wot = w_other.T
    bs = b_self[None, :]
    bo = b_other[None, :]
    gi = gather_index.astype(jnp.int32)
    ri = reduce_index.astype(jnp.int32)
    gidx = jnp.stack([gi, gi + N_NODES]).reshape(NC * N_EDGES)
    ridx = ri

    fst, fot = _tc_matmuls(ft, wst, wot, bs, bo)
    fot_flat = fot.reshape(NC * N_NODES, H)
    acc = _sc_aggregate(fot_flat, fst, gidx, ridx)[:, :N_NODES]
    return jnp.transpose(acc, (0, 2, 1)).reshape(1, D, N_NODES)


# speedup vs baseline: 7.8053x; 2.3349x over previous
import jax
import jax.numpy as jnp
from jax import lax
from jax.experimental import pallas as pl
from jax.experimental.pallas import tpu as pltpu
from jax.experimental.pallas import tpu_sc as plsc

N_NODES = 10000
N_EDGES = 160000
D = 256
H = 128
NC = 2
NS = 16
CHUNK = 40
NCHUNK = 250
ET = NCHUNK * CHUNK
NP = 10240
ROWS_PER_TILE = NP // NS
NPIPE = 5

NB = 2000
GRID = N_NODES // NB


def _matmul_body(ft_ref, ws_ref, wo_ref, bs_ref, bo_ref, fst_ref, fo0_ref,
                 fo1_ref):
    x = ft_ref[...]
    ys = jnp.dot(x, ws_ref[...], preferred_element_type=jnp.float32)
    ys = ys + bs_ref[...]
    yo = jnp.dot(x, wo_ref[...], preferred_element_type=jnp.float32)
    yo = yo + bo_ref[...]
    fst_ref[0] = ys[:, :H]
    fst_ref[1] = ys[:, H:]
    fo0_ref[...] = yo[:, :H]
    fo1_ref[...] = yo[:, H:]


def _tc_matmuls(ft, wst, wot, bs, bo):
    return pl.pallas_call(
        _matmul_body,
        grid=(GRID,),
        in_specs=[
            pl.BlockSpec((NB, D), lambda i: (i, 0)),
            pl.BlockSpec((D, D), lambda i: (0, 0)),
            pl.BlockSpec((D, D), lambda i: (0, 0)),
            pl.BlockSpec((1, D), lambda i: (0, 0)),
            pl.BlockSpec((1, D), lambda i: (0, 0)),
        ],
        out_specs=[
            pl.BlockSpec((NC, NB, H), lambda i: (0, i, 0)),
            pl.BlockSpec((NB, H), lambda i: (i, 0)),
            pl.BlockSpec((NB, H), lambda i: (i, 0)),
        ],
        out_shape=[
            jax.ShapeDtypeStruct((NC, NP, H), jnp.float32),
            jax.ShapeDtypeStruct((NP, H), jnp.float32),
            jax.ShapeDtypeStruct((NP, H), jnp.float32),
        ],
    )(ft, wst, wot, bs, bo)


def _sc_body(fot0, fot1, fst, gidx, ridx, out, idxg_v, acc_sh, rows, r80,
             semg, semr, sems):
    c = lax.axis_index("c")
    s = lax.axis_index("s")
    r0 = s * ROWS_PER_TILE
    pltpu.sync_copy(fst.at[pl.ds(c * NP + r0, ROWS_PER_TILE)],
                    acc_sh.at[pl.ds(r0, ROWS_PER_TILE)])
    pltpu.sync_copy(gidx.at[pl.ds(s * ET, ET)], idxg_v)
    plsc.subcore_barrier()
    r_base = s * ET
    blk = NPIPE * CHUNK

    def scatter_wait(k):
        pltpu.make_async_copy(rows[k], acc_sh.at[r80[k]], sems[k]).wait()

    def run_edges(fot):
        def edge_block(t, carry):
            dr, dg = [], []
            for k in range(NPIPE):
                @pl.when(t > 0)
                def _():
                    scatter_wait(k)
                dr.append(pltpu.async_copy(
                    ridx.at[pl.ds(r_base + t * blk + k * CHUNK, CHUNK)],
                    r80[k], semr[k]))
                dg.append(pltpu.async_copy(
                    fot.at[idxg_v.at[pl.ds(t * blk + k * CHUNK, CHUNK)]],
                    rows[k], semg[k]))
            for k in range(NPIPE):
                dg[k].wait()
                dr[k].wait()
                pltpu.async_copy(rows[k], acc_sh.at[r80[k]], sems[k],
                                 add=True)
            return carry

        lax.fori_loop(0, NCHUNK // NPIPE, edge_block, 0)

    @pl.when(c == 0)
    def _():
        run_edges(fot0)

    @pl.when(c == 1)
    def _():
        run_edges(fot1)

    for k in range(NPIPE):
        scatter_wait(k)
    plsc.subcore_barrier()
    pltpu.sync_copy(acc_sh.at[pl.ds(r0, ROWS_PER_TILE)],
                    out.at[pl.ds(c * NP + r0, ROWS_PER_TILE)])


def _sc_aggregate(fot0, fot1, fst, gidx, ridx):
    mesh = plsc.VectorSubcoreMesh(core_axis_name="c", subcore_axis_name="s")
    return pl.kernel(
        _sc_body,
        out_type=jax.ShapeDtypeStruct((NC * NP, H), jnp.float32),
        mesh=mesh,
        scratch_types=[
            pltpu.VMEM((ET,), jnp.int32),
            pltpu.VMEM_SHARED((NP, H), jnp.float32),
            [pltpu.VMEM((CHUNK, H), jnp.float32)] * NPIPE,
            [pltpu.VMEM((CHUNK,), jnp.int32)] * NPIPE,
            [pltpu.SemaphoreType.DMA] * NPIPE,
            [pltpu.SemaphoreType.DMA] * NPIPE,
            [pltpu.SemaphoreType.DMA] * NPIPE,
        ],
    )(fot0, fot1, fst, gidx, ridx)


def _transpose_body(acc_ref, out_ref):
    out_ref[0, :H, :] = jnp.transpose(acc_ref[0, :N_NODES, :])
    out_ref[0, H:, :] = jnp.transpose(acc_ref[1, :N_NODES, :])


def _tc_assemble(acc3):
    return pl.pallas_call(
        _transpose_body,
        out_shape=jax.ShapeDtypeStruct((1, D, N_NODES), jnp.float32),
    )(acc3)


def kernel(features, w_self, b_self, w_other, b_other, reduce_index,
           gather_index):
    ft = features[0].T
    wst = w_self.T
    wot = w_other.T
    bs = b_self[None, :]
    bo = b_other[None, :]
    gi = gather_index.astype(jnp.int32)
    ri = reduce_index.astype(jnp.int32)

    fst, fot0, fot1 = _tc_matmuls(ft, wst, wot, bs, bo)
    acc = _sc_aggregate(fot0, fot1, fst.reshape(NC * NP, H), gi, ri)
    return _tc_assemble(acc.reshape(NC, NP, H))
